# Initial kernel scaffold; baseline (speedup 1.0000x reference)
import jax
import jax.numpy as jnp
from jax.experimental import pallas as pl


def kernel(pred_bboxes, pred_scores, priors, gt_labels, gt_bboxes, pad_bbox_flag):
    raise NotImplementedError("write your pallas kernel here")


# speedup vs baseline: 39.5376x; 39.5376x over previous
import jax
import jax.numpy as jnp
from jax.experimental import pallas as pl

_EPS = 1e-07
_INF = 100000000.0
_NUM_CLASSES = 80
_RADIUS = 3.0
_TOPK = 13
_IOU_WEIGHT = 3.0
_IMAX = 2147483647


def _assign_body(scores_ref, pbox_ref, prior_ref, gtb_ref, gtl_ref, pad_ref,
                 lbl_ref, met_ref, box_ref):
    G = gtb_ref.shape[1]
    N = scores_ref.shape[2]

    scores = scores_ref[0]
    px1 = pbox_ref[0, 0:1, :]
    py1 = pbox_ref[0, 1:2, :]
    px2 = pbox_ref[0, 2:3, :]
    py2 = pbox_ref[0, 3:4, :]
    pcx = prior_ref[0:1, :]
    pcy = prior_ref[1:2, :]
    pstr = prior_ref[2:3, :]
    gx1 = gtb_ref[0, :, 0:1]
    gy1 = gtb_ref[0, :, 1:2]
    gx2 = gtb_ref[0, :, 2:3]
    gy2 = gtb_ref[0, :, 3:4]
    glab = gtl_ref[0]
    pad = pad_ref[0]

    inside = (pcx > gx1) & (pcy > gy1) & (gx2 > pcx) & (gy2 > pcy)
    inside = inside & (pad > 0.0)
    valid = jnp.any(inside, axis=0, keepdims=True)
    validf = valid.astype(jnp.float32)

    gcx = (gx1 + gx2) * 0.5
    gcy = (gy1 + gy2) * 0.5
    dx = pcx - gcx
    dy = pcy - gcy
    dist = jnp.sqrt(dx * dx + dy * dy) / pstr
    dist = dist * validf
    soft = jnp.exp((dist - _RADIUS) * jnp.float32(2.302585092994046))

    iw = jnp.clip(jnp.minimum(px2, gx2) - jnp.maximum(px1, gx1), 0.0, None)
    ih = jnp.clip(jnp.minimum(py2, gy2) - jnp.maximum(py1, gy1), 0.0, None)
    overlap = iw * ih
    a1 = jnp.clip(px2 - px1, 0.0, None) * jnp.clip(py2 - py1, 0.0, None)
    a2 = jnp.clip(gx2 - gx1, 0.0, None) * jnp.clip(gy2 - gy1, 0.0, None)
    union = a1 + a2 - overlap
    iou = overlap / jnp.maximum(union, 1e-06)
    iou_cost = -jnp.log(iou + _EPS) * _IOU_WEIGHT

    cls_iota = jax.lax.broadcasted_iota(jnp.int32, (G, _NUM_CLASSES), 1)
    onehot_lbl = (cls_iota == glab).astype(jnp.float32)
    p = jax.lax.dot_general(onehot_lbl, scores, (((1,), (0,)), ((), ())),
                            preferred_element_type=jnp.float32)
    sig = 1.0 / (1.0 + jnp.exp(-p))
    sf = iou - sig
    bce = jnp.maximum(p, 0.0) - p * iou + jnp.log1p(jnp.exp(-jnp.abs(p)))
    cls_cost = bce * (sf * sf)

    cost = cls_cost + iou_cost + soft
    cost = jnp.where(valid, cost, _INF)

    iota_n = jax.lax.broadcasted_iota(jnp.int32, (G, N), 1)
    mi = iou
    acc = jnp.zeros((G, 1), jnp.float32)
    for _ in range(_TOPK):
        m = jnp.max(mi, axis=1, keepdims=True)
        acc = acc + m
        first = jnp.min(jnp.where(mi == m, iota_n, N), axis=1, keepdims=True)
        mi = jnp.where(iota_n == first, -1.0, mi)
    ks = jnp.maximum(acc.astype(jnp.int32), 1)
    ks = jnp.where(pad > 0.0, ks, 0)

    ikey = jax.lax.bitcast_convert_type(cost, jnp.int32)
    masked = ikey
    sel = jnp.zeros((G, N), jnp.bool_)
    for t in range(_TOPK):
        m = jnp.min(masked, axis=1, keepdims=True)
        first = jnp.min(jnp.where(masked == m, iota_n, N), axis=1, keepdims=True)
        hit = (iota_n == first) & (t < ks)
        sel = sel | hit
        masked = jnp.where(hit, _IMAX, masked)

    nmatch = jnp.sum(sel.astype(jnp.float32), axis=0, keepdims=True)
    multi = nmatch > 1.0
    cmin = jnp.min(ikey, axis=0, keepdims=True)
    iota_g = jax.lax.broadcasted_iota(jnp.int32, (G, N), 0)
    gmin = jnp.min(jnp.where(ikey == cmin, iota_g, G), axis=0, keepdims=True)
    onehot_g = iota_g == gmin
    matching = (multi & onehot_g) | (jnp.logical_not(multi) & sel)

    fg = jnp.any(matching, axis=0, keepdims=True)
    mf = matching.astype(jnp.float32)
    matched_iou = jnp.sum(mf * iou, axis=0, keepdims=True)
    lab = jnp.sum(jnp.where(matching, glab, 0), axis=0, keepdims=True)
    bx1 = jnp.sum(jnp.where(matching, gx1, 0.0), axis=0, keepdims=True)
    by1 = jnp.sum(jnp.where(matching, gy1, 0.0), axis=0, keepdims=True)
    bx2 = jnp.sum(jnp.where(matching, gx2, 0.0), axis=0, keepdims=True)
    by2 = jnp.sum(jnp.where(matching, gy2, 0.0), axis=0, keepdims=True)

    lbl_ref[0] = jnp.where(fg, lab, _NUM_CLASSES)
    met_ref[0] = jnp.where(fg, matched_iou, 0.0)
    box_ref[0] = jnp.where(fg, jnp.concatenate([bx1, by1, bx2, by2], axis=0), 0.0)


def kernel(pred_bboxes, pred_scores, priors, gt_labels, gt_bboxes, pad_bbox_flag):
    B, N, _ = pred_bboxes.shape
    G = gt_bboxes.shape[1]
    C = pred_scores.shape[2]
    scores_t = jnp.transpose(pred_scores, (0, 2, 1))
    pbox_t = jnp.transpose(pred_bboxes, (0, 2, 1))
    priors_t = jnp.transpose(priors, (1, 0))

    labels_o, metrics_o, boxes_o = pl.pallas_call(
        _assign_body,
        grid=(B,),
        in_specs=[
            pl.BlockSpec((1, C, N), lambda b: (b, 0, 0)),
            pl.BlockSpec((1, 4, N), lambda b: (b, 0, 0)),
            pl.BlockSpec((4, N), lambda b: (0, 0)),
            pl.BlockSpec((1, G, 4), lambda b: (b, 0, 0)),
            pl.BlockSpec((1, G, 1), lambda b: (b, 0, 0)),
            pl.BlockSpec((1, G, 1), lambda b: (b, 0, 0)),
        ],
        out_specs=[
            pl.BlockSpec((1, 1, N), lambda b: (b, 0, 0)),
            pl.BlockSpec((1, 1, N), lambda b: (b, 0, 0)),
            pl.BlockSpec((1, 4, N), lambda b: (b, 0, 0)),
        ],
        out_shape=[
            jax.ShapeDtypeStruct((B, 1, N), jnp.int32),
            jax.ShapeDtypeStruct((B, 1, N), jnp.float32),
            jax.ShapeDtypeStruct((B, 4, N), jnp.float32),
        ],
    )(scores_t, pbox_t, priors_t, gt_bboxes, gt_labels, pad_bbox_flag)

    weights = jnp.ones((B, N), dtype=gt_bboxes.dtype)
    boxes = jnp.transpose(boxes_o, (0, 2, 1))
    return labels_o[:, 0, :], weights, boxes, metrics_o[:, 0, :]


# speedup vs baseline: 42.2901x; 1.0696x over previous
import jax
import jax.numpy as jnp
from jax.experimental import pallas as pl

_EPS = 1e-07
_INF = 100000000.0
_NUM_CLASSES = 80
_RADIUS = 3.0
_TOPK = 13
_IOU_WEIGHT = 3.0
_IMAX = 2147483647


def _assign_body(scores_ref, pbox_ref, prior_ref, gtb_ref, gtl_ref, pad_ref,
                 lbl_ref, met_ref, box_ref):
    G = gtb_ref.shape[1]
    N = scores_ref.shape[2]

    scores = scores_ref[0]
    px1 = pbox_ref[0, 0:1, :]
    py1 = pbox_ref[0, 1:2, :]
    px2 = pbox_ref[0, 2:3, :]
    py2 = pbox_ref[0, 3:4, :]
    pcx = prior_ref[0:1, :]
    pcy = prior_ref[1:2, :]
    pstr = prior_ref[2:3, :]
    gx1 = gtb_ref[0, :, 0:1]
    gy1 = gtb_ref[0, :, 1:2]
    gx2 = gtb_ref[0, :, 2:3]
    gy2 = gtb_ref[0, :, 3:4]
    glab = gtl_ref[0]
    pad = pad_ref[0]

    inside = (pcx > gx1) & (pcy > gy1) & (gx2 > pcx) & (gy2 > pcy)
    inside = inside & (pad > 0.0)
    valid = jnp.any(inside, axis=0, keepdims=True)
    validf = valid.astype(jnp.float32)

    gcx = (gx1 + gx2) * 0.5
    gcy = (gy1 + gy2) * 0.5
    dx = pcx - gcx
    dy = pcy - gcy
    dist = jnp.sqrt(dx * dx + dy * dy) / pstr
    dist = dist * validf
    soft = jnp.exp((dist - _RADIUS) * jnp.float32(2.302585092994046))

    iw = jnp.clip(jnp.minimum(px2, gx2) - jnp.maximum(px1, gx1), 0.0, None)
    ih = jnp.clip(jnp.minimum(py2, gy2) - jnp.maximum(py1, gy1), 0.0, None)
    overlap = iw * ih
    a1 = jnp.clip(px2 - px1, 0.0, None) * jnp.clip(py2 - py1, 0.0, None)
    a2 = jnp.clip(gx2 - gx1, 0.0, None) * jnp.clip(gy2 - gy1, 0.0, None)
    union = a1 + a2 - overlap
    iou = overlap / jnp.maximum(union, 1e-06)
    iou_cost = -jnp.log(iou + _EPS) * _IOU_WEIGHT

    cls_iota = jax.lax.broadcasted_iota(jnp.int32, (G, _NUM_CLASSES), 1)
    onehot_lbl = (cls_iota == glab).astype(jnp.float32)
    p = jax.lax.dot_general(onehot_lbl, scores, (((1,), (0,)), ((), ())),
                            preferred_element_type=jnp.float32)
    sig = 1.0 / (1.0 + jnp.exp(-p))
    sf = iou - sig
    bce = jnp.maximum(p, 0.0) - p * iou + jnp.log1p(jnp.exp(-jnp.abs(p)))
    cls_cost = bce * (sf * sf)

    cost = cls_cost + iou_cost + soft
    cost = jnp.where(valid, cost, _INF)

    iota_n = jax.lax.broadcasted_iota(jnp.int32, (G, N), 1)
    mi = iou
    for _ in range(_TOPK):
        m = jnp.max(mi, axis=1, keepdims=True)
        first = jnp.min(jnp.where(mi == m, iota_n, N), axis=1, keepdims=True)
        mi = jnp.where(iota_n == first, -1.0, mi)
    acc = jnp.sum(jnp.where(mi == -1.0, iou, 0.0), axis=1, keepdims=True)
    ks = jnp.maximum(acc.astype(jnp.int32), 1)
    ks = jnp.where(pad > 0.0, ks, 0)

    ikey = jax.lax.bitcast_convert_type(cost, jnp.int32)
    masked = ikey
    for t in range(_TOPK):
        m = jnp.min(masked, axis=1, keepdims=True)
        first = jnp.min(jnp.where(masked == m, iota_n, N), axis=1, keepdims=True)
        hit = (iota_n == first) & (t < ks)
        masked = jnp.where(hit, _IMAX, masked)
    sel = masked == _IMAX

    nmatch = jnp.sum(sel.astype(jnp.float32), axis=0, keepdims=True)
    multi = nmatch > 1.0
    cmin = jnp.min(ikey, axis=0, keepdims=True)
    iota_g = jax.lax.broadcasted_iota(jnp.int32, (G, N), 0)
    gmin = jnp.min(jnp.where(ikey == cmin, iota_g, G), axis=0, keepdims=True)
    onehot_g = iota_g == gmin
    matching = (multi & onehot_g) | (jnp.logical_not(multi) & sel)

    fg = jnp.any(matching, axis=0, keepdims=True)
    mf = matching.astype(jnp.float32)
    matched_iou = jnp.sum(mf * iou, axis=0, keepdims=True)
    lab = jnp.sum(jnp.where(matching, glab, 0), axis=0, keepdims=True)
    bx1 = jnp.sum(jnp.where(matching, gx1, 0.0), axis=0, keepdims=True)
    by1 = jnp.sum(jnp.where(matching, gy1, 0.0), axis=0, keepdims=True)
    bx2 = jnp.sum(jnp.where(matching, gx2, 0.0), axis=0, keepdims=True)
    by2 = jnp.sum(jnp.where(matching, gy2, 0.0), axis=0, keepdims=True)

    lbl_ref[0] = jnp.where(fg, lab, _NUM_CLASSES)
    met_ref[0] = jnp.where(fg, matched_iou, 0.0)
    box_ref[0] = jnp.where(fg, jnp.concatenate([bx1, by1, bx2, by2], axis=0), 0.0)


def kernel(pred_bboxes, pred_scores, priors, gt_labels, gt_bboxes, pad_bbox_flag):
    B, N, _ = pred_bboxes.shape
    G = gt_bboxes.shape[1]
    C = pred_scores.shape[2]
    scores_t = jnp.transpose(pred_scores, (0, 2, 1))
    pbox_t = jnp.transpose(pred_bboxes, (0, 2, 1))
    priors_t = jnp.transpose(priors, (1, 0))

    labels_o, metrics_o, boxes_o = pl.pallas_call(
        _assign_body,
        grid=(B,),
        in_specs=[
            pl.BlockSpec((1, C, N), lambda b: (b, 0, 0)),
            pl.BlockSpec((1, 4, N), lambda b: (b, 0, 0)),
            pl.BlockSpec((4, N), lambda b: (0, 0)),
            pl.BlockSpec((1, G, 4), lambda b: (b, 0, 0)),
            pl.BlockSpec((1, G, 1), lambda b: (b, 0, 0)),
            pl.BlockSpec((1, G, 1), lambda b: (b, 0, 0)),
        ],
        out_specs=[
            pl.BlockSpec((1, 1, N), lambda b: (b, 0, 0)),
            pl.BlockSpec((1, 1, N), lambda b: (b, 0, 0)),
            pl.BlockSpec((1, 4, N), lambda b: (b, 0, 0)),
        ],
        out_shape=[
            jax.ShapeDtypeStruct((B, 1, N), jnp.int32),
            jax.ShapeDtypeStruct((B, 1, N), jnp.float32),
            jax.ShapeDtypeStruct((B, 4, N), jnp.float32),
        ],
    )(scores_t, pbox_t, priors_t, gt_bboxes, gt_labels, pad_bbox_flag)

    weights = jnp.ones((B, N), dtype=gt_bboxes.dtype)
    boxes = jnp.transpose(boxes_o, (0, 2, 1))
    return labels_o[:, 0, :], weights, boxes, metrics_o[:, 0, :]


# speedup vs baseline: 97.5094x; 2.3057x over previous
import jax
import jax.numpy as jnp
from jax.experimental import pallas as pl

_EPS = 1e-07
_INF = 100000000.0
_NUM_CLASSES = 80
_RADIUS = 3.0
_TOPK = 13
_IOU_WEIGHT = 3.0
_IMAX = 2147483647
_CH = 128


def kernel(pred_bboxes, pred_scores, priors, gt_labels, gt_bboxes, pad_bbox_flag):
    B, N, _ = pred_bboxes.shape
    G = gt_bboxes.shape[1]
    C = pred_scores.shape[2]
    NP = ((N + _CH - 1) // _CH) * _CH
    NCHUNK = NP // _CH

    scores_t = jnp.transpose(pred_scores, (0, 2, 1))
    pbox_t = jnp.transpose(pred_bboxes, (0, 2, 1))
    priors_t = jnp.transpose(priors, (1, 0))
    if NP != N:
        padn = NP - N
        scores_t = jnp.pad(scores_t, ((0, 0), (0, 0), (0, padn)))
        pbox_t = jnp.pad(pbox_t, ((0, 0), (0, 0), (0, padn)))
        prior_fill = jnp.tile(
            jnp.array([[-1e9], [-1e9], [1.0], [1.0]], jnp.float32), (1, padn))
        priors_t = jnp.concatenate([priors_t, prior_fill], axis=1)

    def body(scores_ref, pbox_ref, prior_ref, gtb_ref, gtl_ref, pad_ref,
             lbl_ref, met_ref, box_ref):
        scores = scores_ref[0]
        px1 = pbox_ref[0, 0:1, :]
        py1 = pbox_ref[0, 1:2, :]
        px2 = pbox_ref[0, 2:3, :]
        py2 = pbox_ref[0, 3:4, :]
        pcx = prior_ref[0:1, :]
        pcy = prior_ref[1:2, :]
        pstr = prior_ref[2:3, :]
        gx1 = gtb_ref[0, :, 0:1]
        gy1 = gtb_ref[0, :, 1:2]
        gx2 = gtb_ref[0, :, 2:3]
        gy2 = gtb_ref[0, :, 3:4]
        glab = gtl_ref[0]
        pad = pad_ref[0]

        lane = jax.lax.broadcasted_iota(jnp.int32, (1, NP), 1)
        real = lane < N

        inside = (pcx > gx1) & (pcy > gy1) & (gx2 > pcx) & (gy2 > pcy)
        inside = inside & (pad > 0.0)
        valid = jnp.any(inside, axis=0, keepdims=True)
        validf = valid.astype(jnp.float32)

        gcx = (gx1 + gx2) * 0.5
        gcy = (gy1 + gy2) * 0.5
        dx = pcx - gcx
        dy = pcy - gcy
        dist = jnp.sqrt(dx * dx + dy * dy) / pstr
        dist = dist * validf
        soft = jnp.exp((dist - _RADIUS) * jnp.float32(2.302585092994046))

        iw = jnp.clip(jnp.minimum(px2, gx2) - jnp.maximum(px1, gx1), 0.0, None)
        ih = jnp.clip(jnp.minimum(py2, gy2) - jnp.maximum(py1, gy1), 0.0, None)
        overlap = iw * ih
        a1 = jnp.clip(px2 - px1, 0.0, None) * jnp.clip(py2 - py1, 0.0, None)
        a2 = jnp.clip(gx2 - gx1, 0.0, None) * jnp.clip(gy2 - gy1, 0.0, None)
        union = a1 + a2 - overlap
        iou = overlap / jnp.maximum(union, 1e-06)
        iou_cost = -jnp.log(iou + _EPS) * _IOU_WEIGHT

        cls_iota = jax.lax.broadcasted_iota(jnp.int32, (G, _NUM_CLASSES), 1)
        onehot_lbl = (cls_iota == glab).astype(jnp.float32)
        p = jax.lax.dot_general(onehot_lbl, scores, (((1,), (0,)), ((), ())),
                                preferred_element_type=jnp.float32)
        sig = 1.0 / (1.0 + jnp.exp(-p))
        sf = iou - sig
        bce = jnp.maximum(p, 0.0) - p * iou + jnp.log1p(jnp.exp(-jnp.abs(p)))
        cls_cost = bce * (sf * sf)

        cost = cls_cost + iou_cost + soft
        cost = jnp.where(valid, cost, _INF)
        cost = jnp.where(real, cost, jnp.float32(jnp.inf))

        iou_l = jnp.where(real, iou, -1.0)
        ladder = [jnp.full((G, _CH), -1.0, jnp.float32) for _ in range(_TOPK)]
        for c in range(NCHUNK):
            x = jax.lax.slice_in_dim(iou_l, c * _CH, (c + 1) * _CH, axis=1)
            for i in range(_TOPK):
                hi = jnp.maximum(ladder[i], x)
                x = jnp.minimum(ladder[i], x)
                ladder[i] = hi
        rem = jnp.concatenate(ladder, axis=1)
        acc = jnp.zeros((G, 1), jnp.float32)
        taken = jnp.zeros((G, 1), jnp.float32)
        for _ in range(_TOPK):
            m = jnp.max(rem, axis=1, keepdims=True)
            eq = rem == m
            cnt = jnp.sum(eq.astype(jnp.float32), axis=1, keepdims=True)
            take = jnp.minimum(cnt, float(_TOPK) - taken)
            acc = acc + m * take
            taken = taken + take
            rem = jnp.where(eq, -1.0, rem)
        ks = jnp.maximum(acc.astype(jnp.int32), 1)
        ks = jnp.where(pad > 0.0, ks, 0)

        ikey = jax.lax.bitcast_convert_type(cost, jnp.int32)
        sladder = [jnp.full((G, _CH), _IMAX, jnp.int32) for _ in range(_TOPK)]
        for c in range(NCHUNK):
            x = jax.lax.slice_in_dim(ikey, c * _CH, (c + 1) * _CH, axis=1)
            for i in range(_TOPK):
                lo = jnp.minimum(sladder[i], x)
                x = jnp.maximum(sladder[i], x)
                sladder[i] = lo
        srem = jnp.concatenate(sladder, axis=1)
        cum = jnp.zeros((G, 1), jnp.int32)
        vstar = jnp.zeros((G, 1), jnp.int32)
        stake = jnp.zeros((G, 1), jnp.int32)
        for _ in range(_TOPK):
            m = jnp.min(srem, axis=1, keepdims=True)
            eq = srem == m
            cnt = jnp.sum(eq.astype(jnp.float32), axis=1,
                          keepdims=True).astype(jnp.int32)
            newcum = cum + cnt
            trig = (cum < ks) & (newcum >= ks)
            vstar = jnp.where(trig, m, vstar)
            stake = jnp.where(trig, ks - cum, stake)
            cum = newcum
            srem = jnp.where(eq, _IMAX, srem)

        tri_c = (jax.lax.broadcasted_iota(jnp.int32, (_CH, _CH), 0)
                 < jax.lax.broadcasted_iota(jnp.int32, (_CH, _CH), 1)
                 ).astype(jnp.float32)
        tri_o = (jax.lax.broadcasted_iota(jnp.int32, (NCHUNK, NCHUNK), 0)
                 < jax.lax.broadcasted_iota(jnp.int32, (NCHUNK, NCHUNK), 1)
                 ).astype(jnp.float32)
        eqf = (ikey == vstar).astype(jnp.float32)
        cs_parts = []
        for c in range(NCHUNK):
            eqc = jax.lax.slice_in_dim(eqf, c * _CH, (c + 1) * _CH, axis=1)
            cs_parts.append(jnp.sum(eqc, axis=1, keepdims=True))
        cs = jnp.concatenate(cs_parts, axis=1)
        offs = jax.lax.dot_general(cs, tri_o, (((1,), (0,)), ((), ())),
                                   preferred_element_type=jnp.float32)
        s_f = stake.astype(jnp.float32)
        sel_parts = []
        for c in range(NCHUNK):
            eqc = jax.lax.slice_in_dim(eqf, c * _CH, (c + 1) * _CH, axis=1)
            ikc = jax.lax.slice_in_dim(ikey, c * _CH, (c + 1) * _CH, axis=1)
            pref = jax.lax.dot_general(eqc, tri_c, (((1,), (0,)), ((), ())),
                                       preferred_element_type=jnp.float32)
            pref = pref + offs[:, c:c + 1]
            sel_parts.append((ikc < vstar) | ((eqc > 0.0) & (pref < s_f)))
        sel = jnp.concatenate(sel_parts, axis=1)

        nmatch = jnp.sum(sel.astype(jnp.float32), axis=0, keepdims=True)
        multi = nmatch > 1.0
        cmin = jnp.min(ikey, axis=0, keepdims=True)
        iota_g = jax.lax.broadcasted_iota(jnp.int32, (G, NP), 0)
        gmin = jnp.min(jnp.where(ikey == cmin, iota_g, G), axis=0, keepdims=True)
        onehot_g = iota_g == gmin
        matching = (multi & onehot_g) | (jnp.logical_not(multi) & sel)

        fg = jnp.any(matching, axis=0, keepdims=True)
        mf = matching.astype(jnp.float32)
        matched_iou = jnp.sum(mf * iou, axis=0, keepdims=True)
        lab = jnp.sum(jnp.where(matching, glab, 0), axis=0, keepdims=True)
        bx1 = jnp.sum(jnp.where(matching, gx1, 0.0), axis=0, keepdims=True)
        by1 = jnp.sum(jnp.where(matching, gy1, 0.0), axis=0, keepdims=True)
        bx2 = jnp.sum(jnp.where(matching, gx2, 0.0), axis=0, keepdims=True)
        by2 = jnp.sum(jnp.where(matching, gy2, 0.0), axis=0, keepdims=True)

        lbl_ref[0] = jnp.where(fg, lab, _NUM_CLASSES)
        met_ref[0] = jnp.where(fg, matched_iou, 0.0)
        box_ref[0] = jnp.where(fg, jnp.concatenate([bx1, by1, bx2, by2], axis=0),
                               0.0)

    labels_o, metrics_o, boxes_o = pl.pallas_call(
        body,
        grid=(B,),
        in_specs=[
            pl.BlockSpec((1, C, NP), lambda b: (b, 0, 0)),
            pl.BlockSpec((1, 4, NP), lambda b: (b, 0, 0)),
            pl.BlockSpec((4, NP), lambda b: (0, 0)),
            pl.BlockSpec((1, G, 4), lambda b: (b, 0, 0)),
            pl.BlockSpec((1, G, 1), lambda b: (b, 0, 0)),
            pl.BlockSpec((1, G, 1), lambda b: (b, 0, 0)),
        ],
        out_specs=[
            pl.BlockSpec((1, 1, NP), lambda b: (b, 0, 0)),
            pl.BlockSpec((1, 1, NP), lambda b: (b, 0, 0)),
            pl.BlockSpec((1, 4, NP), lambda b: (b, 0, 0)),
        ],
        out_shape=[
            jax.ShapeDtypeStruct((B, 1, NP), jnp.int32),
            jax.ShapeDtypeStruct((B, 1, NP), jnp.float32),
            jax.ShapeDtypeStruct((B, 4, NP), jnp.float32),
        ],
    )(scores_t, pbox_t, priors_t, gt_bboxes, gt_labels, pad_bbox_flag)

    weights = jnp.ones((B, N), dtype=gt_bboxes.dtype)
    boxes = jnp.transpose(boxes_o[:, :, :N], (0, 2, 1))
    return labels_o[:, 0, :N], weights, boxes, metrics_o[:, 0, :N]
